# baseline (speedup 1.0000x reference)
import functools

import jax
import jax.numpy as jnp
from jax import lax
from jax.experimental import pallas as pl
from jax.experimental.pallas import tpu as pltpu
from jax.experimental.pallas import tpu_sc as plsc

N_NODES = 10000
N_EDGES = 320000
EF = N_NODES + N_EDGES
HEADS = 3
G_POOL = 64

NC, NS, LANES = 2, 16, 16
NW = NC * NS
NR = 64
RW = 160
NPAD = 10240
K = 64
EPAD = ((EF + K - 1) // K) * K


def _proj_body(apply_elu, x_ref, b_ref, w_ref, asm_ref, adm_ref,
               h_ref, ad_ref):
    x = x_ref[...]
    if apply_elu:
        v = x + b_ref[...]
        x = jnp.where(v > 0, v, jnp.exp(jnp.minimum(v, 0.0)) - 1.0)
    h = jnp.dot(x, w_ref[...], preferred_element_type=jnp.float32)
    a_s = jnp.dot(h, asm_ref[...], preferred_element_type=jnp.float32)
    pad = h_ref.shape[1] - h.shape[1] - LANES
    h_ref[...] = jnp.concatenate(
        [h, a_s, jnp.zeros((h.shape[0], pad), jnp.float32)], axis=1)
    ad_ref[...] = jnp.dot(h, adm_ref[...], preferred_element_type=jnp.float32)


def _tc_project(x, b_prev, W, as_mat, ad_mat, apply_elu, hpw):
    fin = x.shape[1]
    hc = W.shape[1]
    rows = 1024
    grid = (NPAD // rows,)
    return pl.pallas_call(
        functools.partial(_proj_body, apply_elu),
        grid=grid,
        in_specs=[
            pl.BlockSpec((rows, fin), lambda i: (i, 0)),
            pl.BlockSpec((1, fin), lambda i: (0, 0)),
            pl.BlockSpec((fin, hc), lambda i: (0, 0)),
            pl.BlockSpec((hc, LANES), lambda i: (0, 0)),
            pl.BlockSpec((hc, LANES), lambda i: (0, 0)),
        ],
        out_specs=[
            pl.BlockSpec((rows, hpw), lambda i: (i, 0)),
            pl.BlockSpec((rows, LANES), lambda i: (i, 0)),
        ],
        out_shape=[
            jax.ShapeDtypeStruct((NPAD, hpw), jnp.float32),
            jax.ShapeDtypeStruct((NPAD, LANES), jnp.float32),
        ],
    )(x, b_prev, W, as_mat, ad_mat)


def _pool_body(h_ref, b_ref, p_ref, wl_ref, bl_ref, out_ref):
    h = h_ref[...] + b_ref[...]
    p = p_ref[...]
    sums = jnp.dot(p, h, preferred_element_type=jnp.float32)
    counts = jnp.sum(p, axis=1, keepdims=True)
    pooled = sums / jnp.maximum(counts, 1.0)
    pooled = jnp.maximum(pooled, 0.0)
    out_ref[...] = jnp.dot(pooled, wl_ref[...],
                           preferred_element_type=jnp.float32) + bl_ref[...]


def _tc_pool_head(h3, b3, P, Wl_pad, bl_pad):
    hc = h3.shape[1]
    return pl.pallas_call(
        _pool_body,
        out_shape=jax.ShapeDtypeStruct((G_POOL, 128), jnp.float32),
    )(h3, b3, P, Wl_pad, bl_pad)


def _sc_body(hc, hpw, src_hbm, dst_hbm, h_hbm, ad_hbm, m_hbm, off_hbm,
             out_hbm, acc, accd, hbuf, adloc, srcv, dstv, mv, offv,
             sem1, sem3):
    c = hc // HEADS
    csl = c // LANES
    wid = lax.axis_index("s") * NC + lax.axis_index("c")
    iota = lax.broadcasted_iota(jnp.int32, (LANES,), 0)
    hmask = iota < HEADS

    pltpu.sync_copy(m_hbm, mv)
    mvec = mv[...]

    for bank in range(2):
        r = wid * 2 + bank
        rbase = r * RW
        pltpu.sync_copy(ad_hbm.at[pl.ds(rbase * LANES, RW * LANES)], adloc)

        zrow = jnp.zeros((LANES,), jnp.float32)

        def _zero_body(i, carry):
            acc[pl.ds(i * LANES, LANES)] = zrow
            return carry

        lax.fori_loop(0, (RW + 1) * hc // LANES, _zero_body, 0)

        def _zero_d(i, carry):
            accd[pl.ds(i * LANES, LANES)] = zrow
            return carry

        lax.fori_loop(0, RW + 1, _zero_d, 0)

        c0 = 0
        c1 = EPAD // K

        def _chunk_body(ci, carry):
            base_e = ci * K
            pltpu.sync_copy(src_hbm.at[pl.ds(base_e, K)], srcv)
            pltpu.sync_copy(dst_hbm.at[pl.ds(base_e, K)], dstv)
            cp3 = pltpu.async_copy(h_hbm.at[srcv], hbuf, sem3)
            cp3.wait()

            def _group_body(g, gcarry):
                dvv = dstv[pl.ds(g * LANES, LANES)]
                for l in range(LANES):
                    e = g * LANES + l
                    ldst = dvv[l] - rbase

                    @pl.when((ldst >= 0) & (ldst < RW))
                    def _():
                        asv = hbuf[e, pl.ds(hc, LANES)]
                        adv = adloc[pl.ds(ldst * LANES, LANES)]
                        evv = asv + adv
                        evv = jnp.where(evv > 0, evv, 0.2 * evv)
                        exv = jnp.exp(evv - mvec)
                        exv = jnp.where(hmask, exv, 0.0)
                        dof = ldst * LANES
                        accd[pl.ds(dof, LANES)] = (
                            accd[pl.ds(dof, LANES)] + exv)
                        abase = ldst * hc
                        for h in range(HEADS):
                            exh = jnp.full((LANES,), exv[h])
                            hco = h * c

                            def _cc_body(cc, ccarry, exh=exh, hco=hco):
                                colb = hco + cc * LANES
                                hv = hbuf[e, pl.ds(colb, LANES)]
                                off = abase + colb
                                acc[pl.ds(off, LANES)] = (
                                    acc[pl.ds(off, LANES)] + exh * hv)
                                return ccarry

                            lax.fori_loop(0, csl, _cc_body, 0)

                return gcarry

            lax.fori_loop(0, K // LANES, _group_body, 0)
            return carry

        lax.fori_loop(c0, c1, _chunk_body, 0)

        def _norm_body(n, carry):
            nbase = n * hc
            dvv = accd[pl.ds(n * LANES, LANES)]
            recv = jnp.float32(1.0) / jnp.maximum(dvv, 1e-16)
            for h in range(HEADS):
                rec = jnp.full((LANES,), recv[h])
                hco = h * c

                def _nc_body(cc, ccarry, rec=rec, hco=hco):
                    off = nbase + hco + cc * LANES
                    acc[pl.ds(off, LANES)] = acc[pl.ds(off, LANES)] * rec
                    return ccarry

                lax.fori_loop(0, csl, _nc_body, 0)
            return carry

        lax.fori_loop(0, RW, _norm_body, 0)
        pltpu.sync_copy(acc.at[pl.ds(0, RW * hc)],
                        out_hbm.at[pl.ds(rbase * hc, RW * hc)])


def _sc_aggregate(hc, hpw, srcS, dstS, h_tab, ad_flat, m16, off_tab):
    mesh = plsc.VectorSubcoreMesh(core_axis_name="c", subcore_axis_name="s")
    kern = pl.kernel(
        functools.partial(_sc_body, hc, hpw),
        out_type=jax.ShapeDtypeStruct((NPAD * hc,), jnp.float32),
        mesh=mesh,
        scratch_types=[
            pltpu.VMEM(((RW + 1) * hc,), jnp.float32),
            pltpu.VMEM(((RW + 1) * LANES,), jnp.float32),
            pltpu.VMEM((K, hpw), jnp.float32),
            pltpu.VMEM((RW * LANES,), jnp.float32),
            pltpu.VMEM((K,), jnp.int32),
            pltpu.VMEM((K,), jnp.int32),
            pltpu.VMEM((LANES,), jnp.float32),
            pltpu.VMEM((LANES,), jnp.int32),
            pltpu.SemaphoreType.DMA,
            pltpu.SemaphoreType.DMA,
        ],
    )
    return kern(srcS, dstS, h_tab, ad_flat, m16, off_tab)


def _blockdiag(a):
    h, c = a.shape
    m = jnp.zeros((h * c, LANES), jnp.float32)
    for j in range(h):
        m = m.at[j * c:(j + 1) * c, j].set(a[j])
    return m


def _layer(x_pad, b_prev, W, a_s, a_d, srcS, dstS, off_tab, apply_elu):
    as_mat = _blockdiag(a_s)
    ad_mat = _blockdiag(a_d)
    hc = W.shape[1]
    hpw = ((hc + LANES + 127) // 128) * 128
    x = x_pad
    if apply_elu:
        v = x + b_prev[None, :]
        x = jnp.where(v > 0, v, jnp.exp(jnp.minimum(v, 0.0)) - 1.0)
    h = x @ W
    a_sv = h @ as_mat
    ad_pad = h @ ad_mat
    h_tab = jnp.concatenate(
        [h, a_sv, jnp.zeros((NPAD, hpw - hc - LANES), jnp.float32)], axis=1)
    mas = jnp.max(h_tab[:N_NODES, hc:hc + HEADS], axis=0)
    mad = jnp.max(ad_pad[:N_NODES, :HEADS], axis=0)
    msum = mas + mad
    m3 = jnp.where(msum > 0, msum, 0.2 * msum)
    m16 = jnp.zeros((LANES,), jnp.float32).at[:HEADS].set(m3)
    agg = _sc_aggregate(hc, hpw, srcS, dstS, h_tab,
                        ad_pad.reshape(NPAD * LANES), m16, off_tab)
    return agg.reshape(NPAD, hc)


def kernel(x, edge_index, batch, y, W1, a1s, a1d, b1, W2, a2s, a2d, b2,
           W3, a3s, a3d, b3, Wl, bl):
    loops = jnp.arange(N_NODES, dtype=edge_index.dtype)
    src = jnp.concatenate([edge_index[0], loops]).astype(jnp.int32)
    dst = jnp.concatenate([edge_index[1], loops]).astype(jnp.int32)

    pad = EPAD - EF
    srcS = jnp.concatenate([src, jnp.zeros((pad,), jnp.int32)])
    dstS = jnp.concatenate([dst, jnp.full((pad,), 2 * N_NODES, jnp.int32)])
    off_tab = jnp.zeros((NR, LANES), jnp.int32)

    x_pad = jnp.zeros((NPAD, x.shape[1]), jnp.float32).at[:N_NODES].set(x)

    zb = jnp.zeros((x.shape[1],), jnp.float32)
    agg1 = _layer(x_pad, zb, W1, a1s, a1d, srcS, dstS, off_tab, False)
    agg2 = _layer(agg1, b1, W2, a2s, a2d, srcS, dstS, off_tab, True)
    agg3 = _layer(agg2, b2, W3, a3s, a3d, srcS, dstS, off_tab, True)

    batch_pad = jnp.concatenate(
        [batch.astype(jnp.int32),
         jnp.full((NPAD - N_NODES,), G_POOL, jnp.int32)])
    P = (batch_pad[None, :] == jnp.arange(G_POOL, dtype=jnp.int32)[:, None]
         ).astype(jnp.float32)
    Wl_pad = jnp.zeros((Wl.shape[0], 128), jnp.float32).at[:, :1].set(Wl)
    bl_pad = jnp.zeros((1, 128), jnp.float32).at[0, 0].set(bl[0])
    out128 = _tc_pool_head(agg3, b3.reshape(1, -1), P, Wl_pad, bl_pad)
    logits = out128[:, :1]

    t = y.reshape(-1, 1).astype(logits.dtype)
    loss = jnp.mean(jnp.maximum(logits, 0.0) - logits * t
                    + jnp.log1p(jnp.exp(-jnp.abs(logits))))
    out = jax.nn.sigmoid(logits)
    return (out, loss)
